# baseline (speedup 1.0000x reference)
import functools

import jax
import jax.numpy as jnp
from jax.experimental import pallas as pl
from jax.experimental.pallas import tpu as pltpu


def _body(x_ref, j_ref, gn_ref, th_ref, beta_ref, njk_ref, alpha_ref,
          post_ref, z_ref, *, C1, J):
    x = x_ref[...]
    logth = jnp.log(th_ref[...])
    fx = jax.lax.dot_general(
        x, logth, (((1,), (1,)), ((), ())),
        preferred_element_type=jnp.float32)

    logT = jnp.log(alpha_ref[0, 0] * beta_ref[0, :C1][None, :]
                   + njk_ref[...][:, :C1])
    j = j_ref[...]
    onehot = (j == jax.lax.broadcasted_iota(jnp.int32, (1, J), 1)
              ).astype(jnp.float32)
    logprior = jax.lax.dot_general(
        onehot, logT, (((1,), (0,)), ((), ())),
        precision=jax.lax.Precision.HIGHEST,
        preferred_element_type=jnp.float32)

    unnorm = logprior + fx
    m = jnp.max(unnorm, axis=1, keepdims=True)
    lse = m + jnp.log(jnp.sum(jnp.exp(unnorm - m), axis=1, keepdims=True))
    log_post = unnorm - lse
    post_ref[...] = jnp.exp(log_post)

    gn = jnp.clip(gn_ref[...], 1e-6, 1.0 - 1e-6)
    g = -jnp.log(-jnp.log(gn))
    z = jnp.argmax(log_post + g, axis=1).astype(jnp.int32)
    z_ref[...] = z[:, None]


def kernel(x, j_batch, gumbel_noise, theta_probs, beta, njk, alpha):
    N, K = x.shape
    C1 = theta_probs.shape[0]
    J, MAXC = njk.shape
    BN = 5000
    assert N % BN == 0
    grid = (N // BN,)

    j2d = j_batch.astype(jnp.int32).reshape(N, 1)
    beta2d = beta.reshape(1, MAXC)
    alpha2d = jnp.asarray(alpha, jnp.float32).reshape(1, 1)

    post, z2d = pl.pallas_call(
        functools.partial(_body, C1=C1, J=J),
        grid=grid,
        in_specs=[
            pl.BlockSpec((BN, K), lambda i: (i, 0)),
            pl.BlockSpec((BN, 1), lambda i: (i, 0)),
            pl.BlockSpec((BN, C1), lambda i: (i, 0)),
            pl.BlockSpec((C1, K), lambda i: (0, 0)),
            pl.BlockSpec((1, MAXC), lambda i: (0, 0)),
            pl.BlockSpec((J, MAXC), lambda i: (0, 0)),
            pl.BlockSpec((1, 1), lambda i: (0, 0)),
        ],
        out_specs=[
            pl.BlockSpec((BN, C1), lambda i: (i, 0)),
            pl.BlockSpec((BN, 1), lambda i: (i, 0)),
        ],
        out_shape=[
            jax.ShapeDtypeStruct((N, C1), jnp.float32),
            jax.ShapeDtypeStruct((N, 1), jnp.int32),
        ],
        compiler_params=pltpu.CompilerParams(
            dimension_semantics=("parallel",)),
    )(x, j2d, gumbel_noise, theta_probs, beta2d, njk, alpha2d)

    return post, z2d[:, 0]


# speedup vs baseline: 1.0997x; 1.0997x over previous
import functools

import jax
import jax.numpy as jnp
from jax.experimental import pallas as pl
from jax.experimental.pallas import tpu as pltpu


def _body(x_ref, j_ref, gn_ref, th_ref, beta_ref, njk_ref, alpha_ref,
          post_ref, z_ref, *, C1, J):
    x = x_ref[...]
    logth = jnp.log(th_ref[...])
    fx = jax.lax.dot_general(
        x, logth, (((1,), (1,)), ((), ())),
        preferred_element_type=jnp.float32)

    logT = jnp.log(alpha_ref[0, 0] * beta_ref[0, :C1][None, :]
                   + njk_ref[...][:, :C1])
    t_hi = logT.astype(jnp.bfloat16)
    r1 = logT - t_hi.astype(jnp.float32)
    t_lo1 = r1.astype(jnp.bfloat16)
    t_lo2 = (r1 - t_lo1.astype(jnp.float32)).astype(jnp.bfloat16)
    j = j_ref[...]
    onehot = (j == jax.lax.broadcasted_iota(jnp.int32, (1, J), 1)
              ).astype(jnp.bfloat16)
    def _gdot(t):
        return jax.lax.dot_general(
            onehot, t, (((1,), (0,)), ((), ())),
            preferred_element_type=jnp.float32)
    logprior = (_gdot(t_hi) + _gdot(t_lo1)) + _gdot(t_lo2)

    unnorm = logprior + fx
    m = jnp.max(unnorm, axis=1, keepdims=True)
    lse = m + jnp.log(jnp.sum(jnp.exp(unnorm - m), axis=1, keepdims=True))
    log_post = unnorm - lse
    post_ref[...] = jnp.exp(log_post)

    gn = jnp.clip(gn_ref[...], 1e-6, 1.0 - 1e-6)
    g = -jnp.log(-jnp.log(gn))
    z = jnp.argmax(log_post + g, axis=1).astype(jnp.int32)
    z_ref[...] = z[:, None]


def kernel(x, j_batch, gumbel_noise, theta_probs, beta, njk, alpha):
    N, K = x.shape
    C1 = theta_probs.shape[0]
    J, MAXC = njk.shape
    BN = 5000
    assert N % BN == 0
    grid = (N // BN,)

    j2d = j_batch.astype(jnp.int32).reshape(N, 1)
    beta2d = beta.reshape(1, MAXC)
    alpha2d = jnp.asarray(alpha, jnp.float32).reshape(1, 1)

    post, z2d = pl.pallas_call(
        functools.partial(_body, C1=C1, J=J),
        grid=grid,
        in_specs=[
            pl.BlockSpec((BN, K), lambda i: (i, 0)),
            pl.BlockSpec((BN, 1), lambda i: (i, 0)),
            pl.BlockSpec((BN, C1), lambda i: (i, 0)),
            pl.BlockSpec((C1, K), lambda i: (0, 0)),
            pl.BlockSpec((1, MAXC), lambda i: (0, 0)),
            pl.BlockSpec((J, MAXC), lambda i: (0, 0)),
            pl.BlockSpec((1, 1), lambda i: (0, 0)),
        ],
        out_specs=[
            pl.BlockSpec((BN, C1), lambda i: (i, 0)),
            pl.BlockSpec((BN, 1), lambda i: (i, 0)),
        ],
        out_shape=[
            jax.ShapeDtypeStruct((N, C1), jnp.float32),
            jax.ShapeDtypeStruct((N, 1), jnp.int32),
        ],
        compiler_params=pltpu.CompilerParams(
            dimension_semantics=("parallel",)),
    )(x, j2d, gumbel_noise, theta_probs, beta2d, njk, alpha2d)

    return post, z2d[:, 0]
